# baseline (speedup 1.0000x reference)
import jax
import jax.numpy as jnp
from jax.experimental import pallas as pl

_MARGIN = 1.35
_VB = 8192


def _scale_body(lab_ref, x_ref, o_ref):
    j = pl.program_id(0)
    x = x_ref[...]
    cols = jax.lax.broadcasted_iota(jnp.int32, x.shape, 1) + j * _VB
    o_ref[...] = jnp.where(cols == lab_ref[...], x * _MARGIN, x)


def kernel(logits, label):
    b, v = logits.shape
    lab = label.astype(jnp.int32).reshape(b, 1)
    grid = (pl.cdiv(v, _VB),)
    return pl.pallas_call(
        _scale_body,
        grid=grid,
        in_specs=[
            pl.BlockSpec((b, 1), lambda j: (0, 0)),
            pl.BlockSpec((b, _VB), lambda j: (0, j)),
        ],
        out_specs=pl.BlockSpec((b, _VB), lambda j: (0, j)),
        out_shape=jax.ShapeDtypeStruct((b, v), logits.dtype),
    )(lab, logits)
